# baseline (speedup 1.0000x reference)
import functools

import jax
import jax.numpy as jnp
from jax import lax
from jax.experimental import pallas as pl
from jax.experimental.pallas import tpu as pltpu
from jax.experimental.pallas import tpu_sc as plsc

_NC = 2
_NS = 16
_EW = 25
_E_LANES = 320000 // _EW
_E_CH = 1280
_L = 16


def _edge_chunk_sum(buf, accs):
    def body(k, carry):
        out = list(carry)
        col = pl.multiple_of(k * _L, _L)
        for r in range(_L):
            out[r] = out[r] + buf[r, pl.ds(col, _L)]
        return tuple(out)
    return lax.fori_loop(0, _E_CH // _L, body, accs)


def _sc_body(edge_ref, oute_ref, ebuf0, ebuf1, stage_e, sem0, sem1):
    cid = lax.axis_index("c")
    sid = lax.axis_index("s")
    wid = sid * _NC + cid

    @pl.when(wid < _EW)
    def _edge():
        base = wid * _E_LANES
        n_chunks = _E_LANES // _E_CH
        bufs = (ebuf0, ebuf1)
        sems = (sem0, sem1)
        copies = [None, None]
        copies[0] = pltpu.async_copy(
            edge_ref.at[:, pl.ds(base, _E_CH)], ebuf0, sem0)
        accs = tuple(jnp.zeros((_L,), jnp.float32) for _ in range(_L))
        for c in range(n_chunks):
            cur = c % 2
            nxt = 1 - cur
            if c + 1 < n_chunks:
                copies[nxt] = pltpu.async_copy(
                    edge_ref.at[:, pl.ds(base + (c + 1) * _E_CH, _E_CH)],
                    bufs[nxt], sems[nxt])
            copies[cur].wait()
            accs = _edge_chunk_sum(bufs[cur], accs)
        ids = lax.iota(jnp.int32, _L)
        v = jnp.zeros((_L,), jnp.float32)
        for r in range(_L):
            v = jnp.where(ids == r, jnp.sum(accs[r]), v)
        stage_e[...] = v
        pltpu.sync_copy(stage_e, oute_ref.at[pl.ds(wid * _L, _L)])


def _sc_edge_partials(et):
    mesh = plsc.VectorSubcoreMesh(core_axis_name="c", subcore_axis_name="s")
    f = pl.kernel(
        _sc_body,
        out_type=jax.ShapeDtypeStruct((_EW * _L,), jnp.float32),
        mesh=mesh,
        scratch_types=[
            pltpu.VMEM((_L, _E_CH), jnp.float32),
            pltpu.VMEM((_L, _E_CH), jnp.float32),
            pltpu.VMEM((_L,), jnp.float32),
            pltpu.SemaphoreType.DMA,
            pltpu.SemaphoreType.DMA,
        ],
        compiler_params=pltpu.CompilerParams(needs_layout_passes=False),
    )
    return f(et)


def _tc_body(pe_ref, b_ref, g_ref, w1_ref, b1_ref, w2_ref, b2_ref,
             o_ref, acc_n, *, grid, inv_e, inv_n, d_edge, d_global):
    i = pl.program_id(0)
    na = jnp.sum(b_ref[...], axis=0, keepdims=True)

    @pl.when(i == 0)
    def _init():
        acc_n[0:1, :] = na

    @pl.when(i > 0)
    def _acc():
        acc_n[0:1, :] = acc_n[0:1, :] + na

    @pl.when(i == grid - 1)
    def _finish():
        esum = pe_ref[...].reshape(1, _EW, d_edge).sum(axis=1) * inv_e
        nmean = acc_n[0:1, :] * inv_n
        wg = w1_ref[:d_global, :]
        we = w1_ref[d_global:d_global + d_edge, :]
        wn = w1_ref[d_global + d_edge:, :]
        pre = (g_ref[...] @ wg + esum @ we + nmean @ wn
               + b1_ref[...][None, :])
        h = jnp.maximum(pre, 0.0)
        o_ref[...] = h @ w2_ref[...] + b2_ref[...][None, :]


def kernel(node_attr, edge_index, edge_attr, global_attr, W1, b1, W2, b2):
    del edge_index
    n_edges, d_edge = edge_attr.shape
    n_nodes, d_feat = node_attr.shape
    d_global = global_attr.shape[1]
    in_features, latent = W1.shape
    out_features = W2.shape[1]

    et = edge_attr.T
    pe = _sc_edge_partials(et).reshape(1, _EW * d_edge)

    grid = 10
    blk_b = n_nodes // grid

    body = functools.partial(_tc_body, grid=grid, inv_e=1.0 / n_edges,
                             inv_n=1.0 / n_nodes, d_edge=d_edge,
                             d_global=d_global)
    out = pl.pallas_call(
        body,
        grid=(grid,),
        in_specs=[
            pl.BlockSpec((1, _EW * d_edge), lambda i: (0, 0)),
            pl.BlockSpec((blk_b, d_feat), lambda i: (i, 0)),
            pl.BlockSpec((1, d_global), lambda i: (0, 0)),
            pl.BlockSpec((in_features, latent), lambda i: (0, 0)),
            pl.BlockSpec((latent,), lambda i: (0,)),
            pl.BlockSpec((latent, out_features), lambda i: (0, 0)),
            pl.BlockSpec((out_features,), lambda i: (0,)),
        ],
        out_specs=pl.BlockSpec((1, out_features), lambda i: (0, 0)),
        out_shape=jax.ShapeDtypeStruct((1, out_features), jnp.float32),
        scratch_shapes=[pltpu.VMEM((8, d_feat), jnp.float32)],
    )(pe, node_attr, global_attr, W1, b1, W2, b2)
    return out


# speedup vs baseline: 1.2638x; 1.2638x over previous
import functools

import jax
import jax.numpy as jnp
from jax import lax
from jax.experimental import pallas as pl
from jax.experimental.pallas import tpu as pltpu
from jax.experimental.pallas import tpu_sc as plsc

_NC = 2
_NS = 16
_EW = 25
_TC_LANES = 115200
_SC_LANES = 320000 - _TC_LANES
_W_LANES = _SC_LANES // _EW
_E_CH = 1024
_L = 16
_GRID = 10


def _edge_chunk_sum(buf, accs):
    def body(k, carry):
        out = list(carry)
        col = pl.multiple_of(k * _L, _L)
        for r in range(_L):
            out[r] = out[r] + buf[r, pl.ds(col, _L)]
        return tuple(out)
    return lax.fori_loop(0, _E_CH // _L, body, accs)


def _sc_body(edge_ref, oute_ref, ebuf0, ebuf1, stage_e, sem0, sem1):
    cid = lax.axis_index("c")
    sid = lax.axis_index("s")
    wid = sid * _NC + cid

    @pl.when(wid < _EW)
    def _edge():
        base = _TC_LANES + wid * _W_LANES
        n_chunks = _W_LANES // _E_CH
        bufs = (ebuf0, ebuf1)
        sems = (sem0, sem1)
        copies = [None, None]
        copies[0] = pltpu.async_copy(
            edge_ref.at[:, pl.ds(base, _E_CH)], ebuf0, sem0)
        accs = tuple(jnp.zeros((_L,), jnp.float32) for _ in range(_L))
        for c in range(n_chunks):
            cur = c % 2
            nxt = 1 - cur
            if c + 1 < n_chunks:
                copies[nxt] = pltpu.async_copy(
                    edge_ref.at[:, pl.ds(base + (c + 1) * _E_CH, _E_CH)],
                    bufs[nxt], sems[nxt])
            copies[cur].wait()
            accs = _edge_chunk_sum(bufs[cur], accs)
        ids = lax.iota(jnp.int32, _L)
        v = jnp.zeros((_L,), jnp.float32)
        for r in range(_L):
            v = jnp.where(ids == r, jnp.sum(accs[r]), v)
        stage_e[...] = v
        pltpu.sync_copy(stage_e, oute_ref.at[pl.ds(wid * _L, _L)])


def _sc_edge_partials(et):
    mesh = plsc.VectorSubcoreMesh(core_axis_name="c", subcore_axis_name="s")
    f = pl.kernel(
        _sc_body,
        out_type=jax.ShapeDtypeStruct((_EW * _L,), jnp.float32),
        mesh=mesh,
        scratch_types=[
            pltpu.VMEM((_L, _E_CH), jnp.float32),
            pltpu.VMEM((_L, _E_CH), jnp.float32),
            pltpu.VMEM((_L,), jnp.float32),
            pltpu.SemaphoreType.DMA,
            pltpu.SemaphoreType.DMA,
        ],
        compiler_params=pltpu.CompilerParams(needs_layout_passes=False),
    )
    return f(et)


def _tc_partial_body(a_ref, b_ref, o1_ref, o2_ref, acc_e, acc_n,
                     *, grid, d_edge):
    i = pl.program_id(0)
    blk = a_ref.shape[1]
    ea = a_ref[...].reshape(d_edge, blk // 128, 128).sum(axis=1)
    na = jnp.sum(b_ref[...], axis=0, keepdims=True)

    @pl.when(i == 0)
    def _init():
        acc_e[...] = ea
        acc_n[0:1, :] = na

    @pl.when(i > 0)
    def _acc():
        acc_e[...] = acc_e[...] + ea
        acc_n[0:1, :] = acc_n[0:1, :] + na

    @pl.when(i == grid - 1)
    def _fin():
        o1_ref[...] = acc_e[...]
        o2_ref[...] = acc_n[...]


def _combine_body(pe_ref, o1_ref, o2_ref, g_ref, w1_ref, b1_ref, w2_ref,
                  b2_ref, o_ref, *, inv_e, inv_n, d_edge, d_global):
    e_sc = pe_ref[...].reshape(1, _EW, d_edge).sum(axis=1) * inv_e
    s_tc = jnp.sum(o1_ref[...], axis=1, keepdims=True) * inv_e
    nmean = o2_ref[0:1, :] * inv_n
    wg = w1_ref[:d_global, :]
    we = w1_ref[d_global:d_global + d_edge, :]
    wn = w1_ref[d_global + d_edge:, :]
    e_pre = e_sc @ we + lax.dot_general(s_tc, we, (((0,), (0,)), ((), ())))
    pre = g_ref[...] @ wg + e_pre + nmean @ wn + b1_ref[...][None, :]
    h = jnp.maximum(pre, 0.0)
    o_ref[...] = h @ w2_ref[...] + b2_ref[...][None, :]


def kernel(node_attr, edge_index, edge_attr, global_attr, W1, b1, W2, b2):
    del edge_index
    n_edges, d_edge = edge_attr.shape
    n_nodes, d_feat = node_attr.shape
    d_global = global_attr.shape[1]
    in_features, latent = W1.shape
    out_features = W2.shape[1]

    et = edge_attr.T

    pe = _sc_edge_partials(et).reshape(1, _EW * d_edge)

    grid = _GRID
    blk_a = _TC_LANES // grid
    blk_b = n_nodes // grid
    tc_body = functools.partial(_tc_partial_body, grid=grid, d_edge=d_edge)
    o1, o2 = pl.pallas_call(
        tc_body,
        grid=(grid,),
        in_specs=[
            pl.BlockSpec((d_edge, blk_a), lambda i: (0, i)),
            pl.BlockSpec((blk_b, d_feat), lambda i: (i, 0)),
        ],
        out_specs=[
            pl.BlockSpec((d_edge, 128), lambda i: (0, 0)),
            pl.BlockSpec((8, d_feat), lambda i: (0, 0)),
        ],
        out_shape=[
            jax.ShapeDtypeStruct((d_edge, 128), jnp.float32),
            jax.ShapeDtypeStruct((8, d_feat), jnp.float32),
        ],
        scratch_shapes=[pltpu.VMEM((d_edge, 128), jnp.float32),
                        pltpu.VMEM((8, d_feat), jnp.float32)],
    )(et, node_attr)

    comb_body = functools.partial(_combine_body, inv_e=1.0 / n_edges,
                                  inv_n=1.0 / n_nodes, d_edge=d_edge,
                                  d_global=d_global)
    out = pl.pallas_call(
        comb_body,
        grid=(1,),
        in_specs=[
            pl.BlockSpec((1, _EW * d_edge), lambda i: (0, 0)),
            pl.BlockSpec((d_edge, 128), lambda i: (0, 0)),
            pl.BlockSpec((8, d_feat), lambda i: (0, 0)),
            pl.BlockSpec((1, d_global), lambda i: (0, 0)),
            pl.BlockSpec((in_features, latent), lambda i: (0, 0)),
            pl.BlockSpec((latent,), lambda i: (0,)),
            pl.BlockSpec((latent, out_features), lambda i: (0, 0)),
            pl.BlockSpec((out_features,), lambda i: (0,)),
        ],
        out_specs=pl.BlockSpec((1, out_features), lambda i: (0, 0)),
        out_shape=jax.ShapeDtypeStruct((1, out_features), jnp.float32),
    )(pe, o1, o2, global_attr, W1, b1, W2, b2)
    return out


# speedup vs baseline: 1.7896x; 1.4161x over previous
import functools

import jax
import jax.numpy as jnp
from jax import lax
from jax.experimental import pallas as pl
from jax.experimental.pallas import tpu as pltpu

_GRID = 25


def _body(a_ref, b_ref, g_ref, w1_ref, b1_ref, w2_ref, b2_ref,
          o_ref, acc_e, acc_n, *, grid, inv_e, inv_n, d_edge, d_global):
    i = pl.program_id(0)
    blk = a_ref.shape[1]
    ea = a_ref[...].reshape(d_edge, blk // 128, 128).sum(axis=1)
    na = jnp.sum(b_ref[...], axis=0, keepdims=True)

    @pl.when(i == 0)
    def _init():
        acc_e[...] = ea
        acc_n[0:1, :] = na

    @pl.when(i > 0)
    def _acc():
        acc_e[...] = acc_e[...] + ea
        acc_n[0:1, :] = acc_n[0:1, :] + na

    @pl.when(i == grid - 1)
    def _finish():
        s16 = jnp.sum(acc_e[...], axis=1, keepdims=True) * inv_e
        nmean = acc_n[0:1, :] * inv_n
        wg = w1_ref[:d_global, :]
        we = w1_ref[d_global:d_global + d_edge, :]
        wn = w1_ref[d_global + d_edge:, :]
        e_pre = lax.dot_general(s16, we, (((0,), (0,)), ((), ())))
        pre = (g_ref[...] @ wg + e_pre + nmean @ wn + b1_ref[...][None, :])
        h = jnp.maximum(pre, 0.0)
        o_ref[...] = h @ w2_ref[...] + b2_ref[...][None, :]


def kernel(node_attr, edge_index, edge_attr, global_attr, W1, b1, W2, b2):
    del edge_index
    n_edges, d_edge = edge_attr.shape
    n_nodes, d_feat = node_attr.shape
    d_global = global_attr.shape[1]
    in_features, latent = W1.shape
    out_features = W2.shape[1]

    et = edge_attr.T

    grid = _GRID
    blk_a = n_edges // grid
    blk_b = n_nodes // grid

    body = functools.partial(_body, grid=grid, inv_e=1.0 / n_edges,
                             inv_n=1.0 / n_nodes, d_edge=d_edge,
                             d_global=d_global)
    out = pl.pallas_call(
        body,
        grid=(grid,),
        in_specs=[
            pl.BlockSpec((d_edge, blk_a), lambda i: (0, i)),
            pl.BlockSpec((blk_b, d_feat), lambda i: (i, 0)),
            pl.BlockSpec((1, d_global), lambda i: (0, 0)),
            pl.BlockSpec((in_features, latent), lambda i: (0, 0)),
            pl.BlockSpec((latent,), lambda i: (0,)),
            pl.BlockSpec((latent, out_features), lambda i: (0, 0)),
            pl.BlockSpec((out_features,), lambda i: (0,)),
        ],
        out_specs=pl.BlockSpec((1, out_features), lambda i: (0, 0)),
        out_shape=jax.ShapeDtypeStruct((1, out_features), jnp.float32),
        scratch_shapes=[pltpu.VMEM((16, 128), jnp.float32),
                        pltpu.VMEM((8, 128), jnp.float32)],
    )(et, node_attr, global_attr, W1, b1, W2, b2)
    return out


# speedup vs baseline: 3.0762x; 1.7189x over previous
import functools

import jax
import jax.numpy as jnp
from jax import lax
from jax.experimental import pallas as pl
from jax.experimental.pallas import tpu as pltpu

_GRID = 5


def _body(a_ref, b_ref, g_ref, w1_ref, b1_ref, w2_ref, b2_ref,
          o_ref, acc_e, acc_n, *, grid, inv_e, inv_n, d_edge, d_global):
    i = pl.program_id(0)
    blk = a_ref.shape[1]
    ea = a_ref[...].reshape(d_edge, blk // 128, 128).sum(axis=1)
    na = jnp.sum(b_ref[...], axis=0, keepdims=True)

    @pl.when(i == 0)
    def _init():
        acc_e[...] = ea
        acc_n[0:1, :] = na

    @pl.when(i > 0)
    def _acc():
        acc_e[...] = acc_e[...] + ea
        acc_n[0:1, :] = acc_n[0:1, :] + na

    @pl.when(i == grid - 1)
    def _finish():
        s16 = jnp.sum(acc_e[...], axis=1, keepdims=True) * inv_e
        nmean = acc_n[0:1, :] * inv_n
        wg = w1_ref[:d_global, :]
        we = w1_ref[d_global:d_global + d_edge, :]
        wn = w1_ref[d_global + d_edge:, :]
        e_pre = lax.dot_general(s16, we, (((0,), (0,)), ((), ())))
        pre = (g_ref[...] @ wg + e_pre + nmean @ wn + b1_ref[...][None, :])
        h = jnp.maximum(pre, 0.0)
        o_ref[...] = h @ w2_ref[...] + b2_ref[...][None, :]


def kernel(node_attr, edge_index, edge_attr, global_attr, W1, b1, W2, b2):
    del edge_index
    n_edges, d_edge = edge_attr.shape
    n_nodes, d_feat = node_attr.shape
    d_global = global_attr.shape[1]
    in_features, latent = W1.shape
    out_features = W2.shape[1]

    et = edge_attr.T

    grid = _GRID
    blk_a = n_edges // grid
    blk_b = n_nodes // grid

    body = functools.partial(_body, grid=grid, inv_e=1.0 / n_edges,
                             inv_n=1.0 / n_nodes, d_edge=d_edge,
                             d_global=d_global)
    out = pl.pallas_call(
        body,
        grid=(grid,),
        in_specs=[
            pl.BlockSpec((d_edge, blk_a), lambda i: (0, i)),
            pl.BlockSpec((blk_b, d_feat), lambda i: (i, 0)),
            pl.BlockSpec((1, d_global), lambda i: (0, 0)),
            pl.BlockSpec((in_features, latent), lambda i: (0, 0)),
            pl.BlockSpec((latent,), lambda i: (0,)),
            pl.BlockSpec((latent, out_features), lambda i: (0, 0)),
            pl.BlockSpec((out_features,), lambda i: (0,)),
        ],
        out_specs=pl.BlockSpec((1, out_features), lambda i: (0, 0)),
        out_shape=jax.ShapeDtypeStruct((1, out_features), jnp.float32),
        scratch_shapes=[pltpu.VMEM((16, 128), jnp.float32),
                        pltpu.VMEM((8, 128), jnp.float32)],
    )(et, node_attr, global_attr, W1, b1, W2, b2)
    return out
